# baseline (speedup 1.0000x reference)
import functools

import jax
import jax.numpy as jnp
from jax import lax
from jax.experimental import pallas as pl
from jax.experimental.pallas import tpu as pltpu
from jax.experimental.pallas import tpu_sc as plsc

_T = 2048
_B = 8
_H = 128
_WIN = 20
_HOPS = 3
_PREC = lax.Precision.HIGHEST


_C = 128
_K = _T // _C


def _lstm_body(text_f, audio_f, text_b, audio_b, wtf, waf, wtb, wab, wblk,
               bcat, hsf, hsb, gxf, gxb, hc):
    k = pl.program_id(0)

    xt_f = text_f[...].reshape(_C * _B, 512)
    xa_f = audio_f[...].reshape(_C * _B, 128)
    gxf[...] = (jnp.dot(xt_f, wtf[...], precision=_PREC)
                + jnp.dot(xa_f, waf[...], precision=_PREC)).reshape(_C, _B, 512)
    xt_b = text_b[...].reshape(_C * _B, 512)
    xa_b = audio_b[...].reshape(_C * _B, 128)
    gxb[...] = (jnp.dot(xt_b, wtb[...], precision=_PREC)
                + jnp.dot(xa_b, wab[...], precision=_PREC)).reshape(_C, _B, 512)

    @pl.when(k == 0)
    def _():
        hc[...] = jnp.zeros_like(hc)

    h0 = hc[0]
    c0 = hc[1]

    def step(s, carry):
        h, c = carry
        gx2 = jnp.concatenate([gxf[s], gxb[_C - 1 - s]], axis=1)
        g = gx2 + jnp.dot(h, wblk[...], precision=_PREC) + bcat[...]
        i2 = jnp.concatenate([g[:, 0:128], g[:, 512:640]], axis=1)
        f2 = jnp.concatenate([g[:, 128:256], g[:, 640:768]], axis=1)
        g2 = jnp.concatenate([g[:, 256:384], g[:, 768:896]], axis=1)
        o2 = jnp.concatenate([g[:, 384:512], g[:, 896:1024]], axis=1)
        i2 = jax.nn.sigmoid(i2)
        f2 = jax.nn.sigmoid(f2)
        g2 = jnp.tanh(g2)
        o2 = jax.nn.sigmoid(o2)
        c = f2 * c + i2 * g2
        h = o2 * jnp.tanh(c)
        hsf[s] = h[:, :128]
        hsb[_C - 1 - s] = h[:, 128:]
        return h, c

    h, c = lax.fori_loop(0, _C, step, (h0, c0))
    hc[0] = h
    hc[1] = c


def _lstm(text_tm, audio_tm, wtf, waf, wtb, wab, wblk, bcat):
    grid = (_K,)
    full = lambda *_: tuple(0 for _ in range(2))
    specs = [
        pl.BlockSpec((_C, _B, 512), lambda k: (k, 0, 0)),
        pl.BlockSpec((_C, _B, 128), lambda k: (k, 0, 0)),
        pl.BlockSpec((_C, _B, 512), lambda k: (_K - 1 - k, 0, 0)),
        pl.BlockSpec((_C, _B, 128), lambda k: (_K - 1 - k, 0, 0)),
        pl.BlockSpec((512, 512), lambda k: (0, 0)),
        pl.BlockSpec((128, 512), lambda k: (0, 0)),
        pl.BlockSpec((512, 512), lambda k: (0, 0)),
        pl.BlockSpec((128, 512), lambda k: (0, 0)),
        pl.BlockSpec((256, 1024), lambda k: (0, 0)),
        pl.BlockSpec((1, 1024), lambda k: (0, 0)),
    ]
    out_specs = [
        pl.BlockSpec((_C, _B, 128), lambda k: (k, 0, 0)),
        pl.BlockSpec((_C, _B, 128), lambda k: (_K - 1 - k, 0, 0)),
    ]
    return pl.pallas_call(
        _lstm_body,
        grid=grid,
        in_specs=specs,
        out_specs=out_specs,
        out_shape=[
            jax.ShapeDtypeStruct((_T, _B, 128), jnp.float32),
            jax.ShapeDtypeStruct((_T, _B, 128), jnp.float32),
        ],
        scratch_shapes=[
            pltpu.VMEM((_C, _B, 512), jnp.float32),
            pltpu.VMEM((_C, _B, 512), jnp.float32),
            pltpu.VMEM((2, _B, 256), jnp.float32),
        ],
    )(text_tm, audio_tm, text_tm, audio_tm, wtf, waf, wtb, wab, wblk, bcat)


_PAD = 32
_RT = 256


def _win21(src_ref, base):
    e0 = base - 24
    s1 = src_ref[pl.ds(e0, _RT + 24), :]
    def dbl(a, st, n):
        return a[n:] + a[:a.shape[0] - n], st + n
    s2, st2 = dbl(s1, e0, 1)
    s4, st4 = dbl(s2, st2, 2)
    s8, st8 = dbl(s4, st4, 4)
    s16, st16 = dbl(s8, st8, 8)
    def at(a, st, row0):
        i = row0 - st
        return a[i:i + _RT]
    return (at(s16, st16, base)
            + at(s4, st4, base - 16)
            + at(s1, e0, base - 20))


def _grn_body(hsf_b, hsb_b, idg, out, cur, nxt, acc):
    ntile = _T // _RT
    cur[pl.ds(0, _PAD), :] = jnp.zeros((_PAD, 256), jnp.float32)
    nxt[pl.ds(0, _PAD), :] = jnp.zeros((_PAD, 256), jnp.float32)
    for rt in range(ntile):
        r = rt * _RT
        x = jnp.concatenate([hsf_b[pl.ds(r, _RT), 0, 0, :],
                             hsb_b[pl.ds(r, _RT), 0, 0, :]], axis=1)
        cur[pl.ds(_PAD + r, _RT), :] = x
        acc[pl.ds(r, _RT), :] = x
    src, dst = cur, nxt
    for _ in range(_HOPS):
        for rt in range(ntile):
            r = rt * _RT
            w = _win21(src, _PAD + r)
            nv = w * idg[pl.ds(r, _RT), :]
            dst[pl.ds(_PAD + r, _RT), :] = nv
            acc[pl.ds(r, _RT), :] = acc[pl.ds(r, _RT), :] + nv
        src, dst = dst, src
    for rt in range(ntile):
        r = rt * _RT
        out[pl.ds(r, _RT), 0, 0, :] = acc[pl.ds(r, _RT), :] * 0.25


def _grn(hsf, hsb, idg):
    out = pl.pallas_call(
        _grn_body,
        grid=(_B,),
        in_specs=[
            pl.BlockSpec((_T, 1, 1, 128), lambda b: (0, b, 0, 0)),
            pl.BlockSpec((_T, 1, 1, 128), lambda b: (0, b, 0, 0)),
            pl.BlockSpec((_T, 1), lambda b: (0, 0)),
        ],
        out_specs=pl.BlockSpec((_T, 1, 1, 256), lambda b: (0, b, 0, 0)),
        out_shape=jax.ShapeDtypeStruct((_T, _B, 1, 256), jnp.float32),
        scratch_shapes=[
            pltpu.VMEM((_T + _PAD, 256), jnp.float32),
            pltpu.VMEM((_T + _PAD, 256), jnp.float32),
            pltpu.VMEM((_T, 256), jnp.float32),
        ],
    )(hsf.reshape(_T, _B, 1, 128), hsb.reshape(_T, _B, 1, 128), idg)
    return out.reshape(_T, _B, 256)


_CH = 256
_NCH = _T // _CH
_HALO = 60
_ROWS = _CH + _HALO
_NLANE = 8
_IDGN = 336


def _sc_hop(src, dst, accb, idgv, h):
    r_lo = 20 * (h + 1)
    nj = _NLANE

    def win_init(r, s):
        return tuple(s[j] + src[r, 0, pl.ds(16 * j, 16)] for j in range(nj))

    s = lax.fori_loop(r_lo - 20, r_lo,
                      win_init,
                      tuple(jnp.zeros((16,), jnp.float32) for _ in range(nj)))

    def mk_body(with_acc):
        def body(r, s):
            news = [s[j] + src[r, 0, pl.ds(16 * j, 16)] for j in range(nj)]
            idg = idgv[pl.ds(r, 16)][0]
            for j in range(nj):
                cur = news[j] * idg
                dst[r, 0, pl.ds(16 * j, 16)] = cur
                if with_acc:
                    a = accb[r - _HALO, 0, pl.ds(16 * j, 16)] + cur
                    if h == _HOPS - 1:
                        a = a * (1.0 / (_HOPS + 1))
                    accb[r - _HALO, 0, pl.ds(16 * j, 16)] = a
            return tuple(news[j] - src[r - 20, 0, pl.ds(16 * j, 16)]
                         for j in range(nj))
        return body

    if r_lo < _HALO:
        s = lax.fori_loop(r_lo, _HALO, mk_body(False), s)
    lax.fori_loop(_HALO, _ROWS, mk_body(True), s)


def _grn_sc_body(hsf_hbm, hsb_hbm, idg_hbm, outf_hbm, outb_hbm,
                 abuf, bbuf, accb, idgv):
    wid = lax.axis_index("s") * 2 + lax.axis_index("c")

    for i in range(4):
        src_arr = hsf_hbm if (i % 2 == 0) else hsb_hbm
        out_arr = outf_hbm if (i % 2 == 0) else outb_hbm
        rest = wid + 32 * (i // 2)
        b = rest // _NCH
        tc = rest % _NCH
        start = tc * _CH
        halo0 = jnp.maximum(start - _HALO, 0)

        pltpu.sync_copy(idg_hbm.at[pl.ds(start, _IDGN)], idgv)

        @pl.when(tc > 0)
        def _():
            pltpu.sync_copy(
                src_arr.at[pl.ds(halo0, _HALO), pl.ds(b, 1), :],
                abuf.at[pl.ds(0, _HALO)])

        @pl.when(tc == 0)
        def _():
            def zrow(r, c):
                for j in range(_NLANE):
                    abuf[r, 0, pl.ds(16 * j, 16)] = jnp.zeros((16,),
                                                              jnp.float32)
                return c
            lax.fori_loop(0, _HALO, zrow, 0)

        pltpu.sync_copy(
            src_arr.at[pl.ds(start, _CH), pl.ds(b, 1), :],
            abuf.at[pl.ds(_HALO, _CH)])
        pltpu.sync_copy(
            src_arr.at[pl.ds(start, _CH), pl.ds(b, 1), :], accb)

        _sc_hop(abuf, bbuf, accb, idgv, 0)
        _sc_hop(bbuf, abuf, accb, idgv, 1)
        _sc_hop(abuf, bbuf, accb, idgv, 2)

        pltpu.sync_copy(
            accb, out_arr.at[pl.ds(start, _CH), pl.ds(b, 1), :])


def _grn_sc(hsf, hsb, idg_pad):
    mesh = plsc.VectorSubcoreMesh(core_axis_name="c", subcore_axis_name="s")
    f = functools.partial(
        pl.kernel,
        out_type=[
            jax.ShapeDtypeStruct((_T, _B, 128), jnp.float32),
            jax.ShapeDtypeStruct((_T, _B, 128), jnp.float32),
        ],
        mesh=mesh,
        scratch_types=[
            pltpu.VMEM((_ROWS, 1, 128), jnp.float32),
            pltpu.VMEM((_ROWS, 1, 128), jnp.float32),
            pltpu.VMEM((_CH, 1, 128), jnp.float32),
            pltpu.VMEM((_IDGN,), jnp.float32),
        ],
    )(_grn_sc_body)
    return f(hsf, hsb, idg_pad)


_CF = 256


def _fusion_body(hsf, hsb, grf, grb, wg_l, wg_g, bg, wx, wgr, bfv, wc, bc,
                 out):
    n = _CF * _B
    l = jnp.concatenate([hsf[...].reshape(n, 128), hsb[...].reshape(n, 128)],
                        axis=1)
    g = jnp.concatenate([grf[...].reshape(n, 128), grb[...].reshape(n, 128)],
                        axis=1)
    gate = jax.nn.sigmoid(jnp.dot(l, wg_l[...], precision=_PREC)
                          + jnp.dot(g, wg_g[...], precision=_PREC) + bg[...])
    fused = jnp.tanh(gate * jnp.dot(l, wx[...], precision=_PREC)
                     + (1.0 - gate) * jnp.dot(g, wgr[...], precision=_PREC)
                     + bfv[...])
    out[...] = (jnp.dot(fused, wc[...], precision=_PREC)
                + bc[...]).reshape(_CF, _B, 128)


def _fusion(hsf, hsb, grf, grb, wg_l, wg_g, bg, wx, wgr, bfv, wc, bc):
    m = _T // _CF
    wspec = lambda shp: pl.BlockSpec(shp, lambda k: (0, 0))
    return pl.pallas_call(
        _fusion_body,
        grid=(m,),
        in_specs=[
            pl.BlockSpec((_CF, _B, 128), lambda k: (k, 0, 0)),
            pl.BlockSpec((_CF, _B, 128), lambda k: (k, 0, 0)),
            pl.BlockSpec((_CF, _B, 128), lambda k: (k, 0, 0)),
            pl.BlockSpec((_CF, _B, 128), lambda k: (k, 0, 0)),
            wspec((256, 256)), wspec((256, 256)), wspec((1, 256)),
            wspec((256, 256)), wspec((256, 256)), wspec((1, 256)),
            wspec((256, 128)), wspec((1, 128)),
        ],
        out_specs=pl.BlockSpec((_CF, _B, 128), lambda k: (k, 0, 0)),
        out_shape=jax.ShapeDtypeStruct((_T, _B, 128), jnp.float32),
    )(hsf, hsb, grf, grb, wg_l, wg_g, bg, wx, wgr, bfv, wc, bc)


def kernel(text_embeds, audio_feats, speaker_ids, W_ih_f, W_hh_f, b_f,
           W_ih_b, W_hh_b, b_b, Wg, bg, Wx, Wgr, bf, Wc, bc):
    del speaker_ids
    f32 = jnp.float32

    text_tm = jnp.swapaxes(text_embeds, 0, 1)
    audio_tm = jnp.swapaxes(audio_feats, 0, 1)

    wtf = W_ih_f[:, :512].T
    waf = W_ih_f[:, 512:].T
    wtb = W_ih_b[:, :512].T
    wab = W_ih_b[:, 512:].T
    wblk = jnp.zeros((256, 1024), f32)
    wblk = wblk.at[:128, :512].set(W_hh_f.T)
    wblk = wblk.at[128:, 512:].set(W_hh_b.T)
    bcat = jnp.concatenate([b_f, b_b]).reshape(1, 1024)

    hsf, hsb = _lstm(text_tm, audio_tm, wtf, waf, wtb, wab, wblk, bcat)

    p = jnp.arange(_T + 128, dtype=f32)
    idg_pad = 1.0 / jnp.clip(p - _HALO + 1.0, 1.0, 21.0)
    grf, grb = _grn_sc(hsf, hsb, idg_pad)

    wg_l = Wg[:, :256].T
    wg_g = Wg[:, 256:].T
    wc_pad = jnp.zeros((256, 128), f32).at[:, :7].set(Wc.T)
    bc_pad = jnp.zeros((1, 128), f32).at[0, :7].set(bc)
    out = _fusion(hsf, hsb, grf, grb, wg_l, wg_g, bg.reshape(1, 256),
                  Wx.T, Wgr.T, bf.reshape(1, 256), wc_pad, bc_pad)

    return jnp.swapaxes(out[:, :, :7], 0, 1)


# speedup vs baseline: 2.6694x; 2.6694x over previous
import functools

import jax
import jax.numpy as jnp
from jax import lax
from jax.experimental import pallas as pl
from jax.experimental.pallas import tpu as pltpu
from jax.experimental.pallas import tpu_sc as plsc

_T = 2048
_B = 8
_H = 128
_WIN = 20
_HOPS = 3
_PREC = lax.Precision.DEFAULT
_PREC_R = lax.Precision.DEFAULT


_C = 128
_K = _T // _C


def _lstm_body(text_f, audio_f, text_b, audio_b, wtf, waf, wtb, wab, wblk,
               bcat, hsf, hsb, gxf, gxb, hc):
    k = pl.program_id(0)

    xt_f = text_f[...].reshape(_C * _B, 512)
    xa_f = audio_f[...].reshape(_C * _B, 128)
    gxf[...] = (jnp.dot(xt_f, wtf[...], precision=_PREC)
                + jnp.dot(xa_f, waf[...], precision=_PREC)).reshape(_C, _B, 512)
    xt_b = text_b[...].reshape(_C * _B, 512)
    xa_b = audio_b[...].reshape(_C * _B, 128)
    gxb[...] = (jnp.dot(xt_b, wtb[...], precision=_PREC)
                + jnp.dot(xa_b, wab[...], precision=_PREC)).reshape(_C, _B, 512)

    @pl.when(k == 0)
    def _():
        hc[...] = jnp.zeros_like(hc)

    h0 = hc[0]
    c0 = hc[1]

    def step(s, carry):
        h, c = carry
        gx2 = jnp.concatenate([gxf[s], gxb[_C - 1 - s]], axis=1)
        g = gx2 + jnp.dot(h, wblk[...], precision=_PREC_R) + bcat[...]
        i2 = jnp.concatenate([g[:, 0:128], g[:, 512:640]], axis=1)
        f2 = jnp.concatenate([g[:, 128:256], g[:, 640:768]], axis=1)
        g2 = jnp.concatenate([g[:, 256:384], g[:, 768:896]], axis=1)
        o2 = jnp.concatenate([g[:, 384:512], g[:, 896:1024]], axis=1)
        i2 = jax.nn.sigmoid(i2)
        f2 = jax.nn.sigmoid(f2)
        g2 = jnp.tanh(g2)
        o2 = jax.nn.sigmoid(o2)
        c = f2 * c + i2 * g2
        h = o2 * jnp.tanh(c)
        hsf[s] = h[:, :128]
        hsb[_C - 1 - s] = h[:, 128:]
        return h, c

    h, c = lax.fori_loop(0, _C, step, (h0, c0))
    hc[0] = h
    hc[1] = c


def _lstm(text_tm, audio_tm, wtf, waf, wtb, wab, wblk, bcat):
    grid = (_K,)
    full = lambda *_: tuple(0 for _ in range(2))
    specs = [
        pl.BlockSpec((_C, _B, 512), lambda k: (k, 0, 0)),
        pl.BlockSpec((_C, _B, 128), lambda k: (k, 0, 0)),
        pl.BlockSpec((_C, _B, 512), lambda k: (_K - 1 - k, 0, 0)),
        pl.BlockSpec((_C, _B, 128), lambda k: (_K - 1 - k, 0, 0)),
        pl.BlockSpec((512, 512), lambda k: (0, 0)),
        pl.BlockSpec((128, 512), lambda k: (0, 0)),
        pl.BlockSpec((512, 512), lambda k: (0, 0)),
        pl.BlockSpec((128, 512), lambda k: (0, 0)),
        pl.BlockSpec((256, 1024), lambda k: (0, 0)),
        pl.BlockSpec((1, 1024), lambda k: (0, 0)),
    ]
    out_specs = [
        pl.BlockSpec((_C, _B, 128), lambda k: (k, 0, 0)),
        pl.BlockSpec((_C, _B, 128), lambda k: (_K - 1 - k, 0, 0)),
    ]
    return pl.pallas_call(
        _lstm_body,
        grid=grid,
        in_specs=specs,
        out_specs=out_specs,
        out_shape=[
            jax.ShapeDtypeStruct((_T, _B, 128), jnp.float32),
            jax.ShapeDtypeStruct((_T, _B, 128), jnp.float32),
        ],
        scratch_shapes=[
            pltpu.VMEM((_C, _B, 512), jnp.float32),
            pltpu.VMEM((_C, _B, 512), jnp.float32),
            pltpu.VMEM((2, _B, 256), jnp.float32),
        ],
    )(text_tm, audio_tm, text_tm, audio_tm, wtf, waf, wtb, wab, wblk, bcat)


_PAD = 32
_RT = 256


def _win21(src_ref, base):
    e0 = base - 24
    s1 = src_ref[pl.ds(e0, _RT + 24), :]
    def dbl(a, st, n):
        return a[n:] + a[:a.shape[0] - n], st + n
    s2, st2 = dbl(s1, e0, 1)
    s4, st4 = dbl(s2, st2, 2)
    s8, st8 = dbl(s4, st4, 4)
    s16, st16 = dbl(s8, st8, 8)
    def at(a, st, row0):
        i = row0 - st
        return a[i:i + _RT]
    return (at(s16, st16, base)
            + at(s4, st4, base - 16)
            + at(s1, e0, base - 20))


def _grn_body(hsf_b, hsb_b, idg, out, cur, nxt, acc):
    ntile = _T // _RT
    cur[pl.ds(0, _PAD), :] = jnp.zeros((_PAD, 256), jnp.float32)
    nxt[pl.ds(0, _PAD), :] = jnp.zeros((_PAD, 256), jnp.float32)
    for rt in range(ntile):
        r = rt * _RT
        x = jnp.concatenate([hsf_b[pl.ds(r, _RT), 0, 0, :],
                             hsb_b[pl.ds(r, _RT), 0, 0, :]], axis=1)
        cur[pl.ds(_PAD + r, _RT), :] = x
        acc[pl.ds(r, _RT), :] = x
    src, dst = cur, nxt
    for _ in range(_HOPS):
        for rt in range(ntile):
            r = rt * _RT
            w = _win21(src, _PAD + r)
            nv = w * idg[pl.ds(r, _RT), :]
            dst[pl.ds(_PAD + r, _RT), :] = nv
            acc[pl.ds(r, _RT), :] = acc[pl.ds(r, _RT), :] + nv
        src, dst = dst, src
    for rt in range(ntile):
        r = rt * _RT
        out[pl.ds(r, _RT), 0, 0, :] = acc[pl.ds(r, _RT), :] * 0.25


def _grn(hsf, hsb, idg):
    out = pl.pallas_call(
        _grn_body,
        grid=(_B,),
        in_specs=[
            pl.BlockSpec((_T, 1, 1, 128), lambda b: (0, b, 0, 0)),
            pl.BlockSpec((_T, 1, 1, 128), lambda b: (0, b, 0, 0)),
            pl.BlockSpec((_T, 1), lambda b: (0, 0)),
        ],
        out_specs=pl.BlockSpec((_T, 1, 1, 256), lambda b: (0, b, 0, 0)),
        out_shape=jax.ShapeDtypeStruct((_T, _B, 1, 256), jnp.float32),
        scratch_shapes=[
            pltpu.VMEM((_T + _PAD, 256), jnp.float32),
            pltpu.VMEM((_T + _PAD, 256), jnp.float32),
            pltpu.VMEM((_T, 256), jnp.float32),
        ],
    )(hsf.reshape(_T, _B, 1, 128), hsb.reshape(_T, _B, 1, 128), idg)
    return out.reshape(_T, _B, 256)


_CH = 256
_NCH = _T // _CH
_HALO = 60
_ROWS = _CH + _HALO
_NLANE = 8
_IDGN = 336


def _sc_hop(src, dst, accb, idgv, h):
    r_lo = 20 * (h + 1)
    nj = _NLANE

    def win_init(r, s):
        return tuple(s[j] + src[r, 0, pl.ds(16 * j, 16)] for j in range(nj))

    s = lax.fori_loop(r_lo - 20, r_lo,
                      win_init,
                      tuple(jnp.zeros((16,), jnp.float32) for _ in range(nj)))

    def mk_body(with_acc):
        def body(r, s):
            news = [s[j] + src[r, 0, pl.ds(16 * j, 16)] for j in range(nj)]
            idg = idgv[pl.ds(r, 16)][0]
            for j in range(nj):
                cur = news[j] * idg
                dst[r, 0, pl.ds(16 * j, 16)] = cur
                if with_acc:
                    a = accb[r - _HALO, 0, pl.ds(16 * j, 16)] + cur
                    if h == _HOPS - 1:
                        a = a * (1.0 / (_HOPS + 1))
                    accb[r - _HALO, 0, pl.ds(16 * j, 16)] = a
            return tuple(news[j] - src[r - 20, 0, pl.ds(16 * j, 16)]
                         for j in range(nj))
        return body

    if r_lo < _HALO:
        s = lax.fori_loop(r_lo, _HALO, mk_body(False), s)
    lax.fori_loop(_HALO, _ROWS, mk_body(True), s)


def _grn_sc_body(hsf_hbm, hsb_hbm, idg_hbm, outf_hbm, outb_hbm,
                 abuf, bbuf, accb, idgv):
    wid = lax.axis_index("s") * 2 + lax.axis_index("c")

    for i in range(4):
        src_arr = hsf_hbm if (i % 2 == 0) else hsb_hbm
        out_arr = outf_hbm if (i % 2 == 0) else outb_hbm
        rest = wid + 32 * (i // 2)
        b = rest // _NCH
        tc = rest % _NCH
        start = tc * _CH
        halo0 = jnp.maximum(start - _HALO, 0)

        pltpu.sync_copy(idg_hbm.at[pl.ds(start, _IDGN)], idgv)

        @pl.when(tc > 0)
        def _():
            pltpu.sync_copy(
                src_arr.at[pl.ds(halo0, _HALO), pl.ds(b, 1), :],
                abuf.at[pl.ds(0, _HALO)])

        @pl.when(tc == 0)
        def _():
            def zrow(r, c):
                for j in range(_NLANE):
                    abuf[r, 0, pl.ds(16 * j, 16)] = jnp.zeros((16,),
                                                              jnp.float32)
                return c
            lax.fori_loop(0, _HALO, zrow, 0)

        pltpu.sync_copy(
            src_arr.at[pl.ds(start, _CH), pl.ds(b, 1), :],
            abuf.at[pl.ds(_HALO, _CH)])
        pltpu.sync_copy(
            src_arr.at[pl.ds(start, _CH), pl.ds(b, 1), :], accb)

        _sc_hop(abuf, bbuf, accb, idgv, 0)
        _sc_hop(bbuf, abuf, accb, idgv, 1)
        _sc_hop(abuf, bbuf, accb, idgv, 2)

        pltpu.sync_copy(
            accb, out_arr.at[pl.ds(start, _CH), pl.ds(b, 1), :])


def _grn_sc(hsf, hsb, idg_pad):
    mesh = plsc.VectorSubcoreMesh(core_axis_name="c", subcore_axis_name="s")
    f = functools.partial(
        pl.kernel,
        out_type=[
            jax.ShapeDtypeStruct((_T, _B, 128), jnp.float32),
            jax.ShapeDtypeStruct((_T, _B, 128), jnp.float32),
        ],
        mesh=mesh,
        scratch_types=[
            pltpu.VMEM((_ROWS, 1, 128), jnp.float32),
            pltpu.VMEM((_ROWS, 1, 128), jnp.float32),
            pltpu.VMEM((_CH, 1, 128), jnp.float32),
            pltpu.VMEM((_IDGN,), jnp.float32),
        ],
    )(_grn_sc_body)
    return f(hsf, hsb, idg_pad)


_CF = 256


def _fusion_body(hsf, hsb, grf, grb, wg_l, wg_g, bg, wx, wgr, bfv, wc, bc,
                 out):
    n = _CF * _B
    l = jnp.concatenate([hsf[...].reshape(n, 128), hsb[...].reshape(n, 128)],
                        axis=1)
    g = jnp.concatenate([grf[...].reshape(n, 128), grb[...].reshape(n, 128)],
                        axis=1)
    gate = jax.nn.sigmoid(jnp.dot(l, wg_l[...], precision=_PREC)
                          + jnp.dot(g, wg_g[...], precision=_PREC) + bg[...])
    fused = jnp.tanh(gate * jnp.dot(l, wx[...], precision=_PREC)
                     + (1.0 - gate) * jnp.dot(g, wgr[...], precision=_PREC)
                     + bfv[...])
    out[...] = (jnp.dot(fused, wc[...], precision=_PREC)
                + bc[...]).reshape(_CF, _B, 128)


def _fusion(hsf, hsb, grf, grb, wg_l, wg_g, bg, wx, wgr, bfv, wc, bc):
    m = _T // _CF
    wspec = lambda shp: pl.BlockSpec(shp, lambda k: (0, 0))
    return pl.pallas_call(
        _fusion_body,
        grid=(m,),
        in_specs=[
            pl.BlockSpec((_CF, _B, 128), lambda k: (k, 0, 0)),
            pl.BlockSpec((_CF, _B, 128), lambda k: (k, 0, 0)),
            pl.BlockSpec((_CF, _B, 128), lambda k: (k, 0, 0)),
            pl.BlockSpec((_CF, _B, 128), lambda k: (k, 0, 0)),
            wspec((256, 256)), wspec((256, 256)), wspec((1, 256)),
            wspec((256, 256)), wspec((256, 256)), wspec((1, 256)),
            wspec((256, 128)), wspec((1, 128)),
        ],
        out_specs=pl.BlockSpec((_CF, _B, 128), lambda k: (k, 0, 0)),
        out_shape=jax.ShapeDtypeStruct((_T, _B, 128), jnp.float32),
    )(hsf, hsb, grf, grb, wg_l, wg_g, bg, wx, wgr, bfv, wc, bc)


def kernel(text_embeds, audio_feats, speaker_ids, W_ih_f, W_hh_f, b_f,
           W_ih_b, W_hh_b, b_b, Wg, bg, Wx, Wgr, bf, Wc, bc):
    del speaker_ids
    f32 = jnp.float32

    text_tm = jnp.swapaxes(text_embeds, 0, 1)
    audio_tm = jnp.swapaxes(audio_feats, 0, 1)

    wtf = W_ih_f[:, :512].T
    waf = W_ih_f[:, 512:].T
    wtb = W_ih_b[:, :512].T
    wab = W_ih_b[:, 512:].T
    wblk = jnp.zeros((256, 1024), f32)
    wblk = wblk.at[:128, :512].set(W_hh_f.T)
    wblk = wblk.at[128:, 512:].set(W_hh_b.T)
    bcat = jnp.concatenate([b_f, b_b]).reshape(1, 1024)

    hsf, hsb = _lstm(text_tm, audio_tm, wtf, waf, wtb, wab, wblk, bcat)

    p = jnp.arange(_T + 128, dtype=f32)
    idg_pad = 1.0 / jnp.clip(p - _HALO + 1.0, 1.0, 21.0)
    grf, grb = _grn_sc(hsf, hsb, idg_pad)

    wg_l = Wg[:, :256].T
    wg_g = Wg[:, 256:].T
    wc_pad = jnp.zeros((256, 128), f32).at[:, :7].set(Wc.T)
    bc_pad = jnp.zeros((1, 128), f32).at[0, :7].set(bc)
    out = _fusion(hsf, hsb, grf, grb, wg_l, wg_g, bg.reshape(1, 256),
                  Wx.T, Wgr.T, bf.reshape(1, 256), wc_pad, bc_pad)

    return jnp.swapaxes(out[:, :, :7], 0, 1)


# speedup vs baseline: 2.9363x; 1.1000x over previous
import functools

import jax
import jax.numpy as jnp
from jax import lax
from jax.experimental import pallas as pl
from jax.experimental.pallas import tpu as pltpu
from jax.experimental.pallas import tpu_sc as plsc

_T = 2048
_B = 8
_H = 128
_WIN = 20
_HOPS = 3
_PREC = lax.Precision.DEFAULT
_PREC_R = lax.Precision.DEFAULT


_C = 128
_K = _T // _C


def _lstm_body(text_f, audio_f, text_b, audio_b, wtf, waf, wtb, wab, wblk,
               bcat, hsf, hsb, gxf, gxb, hc):
    k = pl.program_id(0)

    xt_f = text_f[...].reshape(_C * _B, 512)
    xa_f = audio_f[...].reshape(_C * _B, 128)
    gxf[...] = (jnp.dot(xt_f, wtf[...], precision=_PREC)
                + jnp.dot(xa_f, waf[...], precision=_PREC)
                + bcat[:, :512]).reshape(_C, _B, 512)
    xt_b = text_b[...].reshape(_C * _B, 512)
    xa_b = audio_b[...].reshape(_C * _B, 128)
    gxb[...] = (jnp.dot(xt_b, wtb[...], precision=_PREC)
                + jnp.dot(xa_b, wab[...], precision=_PREC)
                + bcat[:, 512:]).reshape(_C, _B, 512)

    @pl.when(k == 0)
    def _():
        hc[...] = jnp.zeros_like(hc)

    h0 = hc[0]
    c0 = hc[1]

    def step(s, carry):
        h, c = carry
        ghh = jnp.dot(h, wblk[...], precision=_PREC_R)
        gf = gxf[s] + ghh[:, :512]
        gb = gxb[_C - 1 - s] + ghh[:, 512:]
        i_f = jax.nn.sigmoid(gf[:, 0:128])
        f_f = jax.nn.sigmoid(gf[:, 128:256])
        g_f = jnp.tanh(gf[:, 256:384])
        o_f = jax.nn.sigmoid(gf[:, 384:512])
        i_b = jax.nn.sigmoid(gb[:, 0:128])
        f_b = jax.nn.sigmoid(gb[:, 128:256])
        g_b = jnp.tanh(gb[:, 256:384])
        o_b = jax.nn.sigmoid(gb[:, 384:512])
        c_f = f_f * c[:, :128] + i_f * g_f
        c_b = f_b * c[:, 128:] + i_b * g_b
        h_f = o_f * jnp.tanh(c_f)
        h_b = o_b * jnp.tanh(c_b)
        hsf[s] = h_f
        hsb[_C - 1 - s] = h_b
        return (jnp.concatenate([h_f, h_b], axis=1),
                jnp.concatenate([c_f, c_b], axis=1))

    h, c = lax.fori_loop(0, _C, step, (h0, c0), unroll=4)
    hc[0] = h
    hc[1] = c


def _lstm(text_tm, audio_tm, wtf, waf, wtb, wab, wblk, bcat):
    grid = (_K,)
    full = lambda *_: tuple(0 for _ in range(2))
    specs = [
        pl.BlockSpec((_C, _B, 512), lambda k: (k, 0, 0)),
        pl.BlockSpec((_C, _B, 128), lambda k: (k, 0, 0)),
        pl.BlockSpec((_C, _B, 512), lambda k: (_K - 1 - k, 0, 0)),
        pl.BlockSpec((_C, _B, 128), lambda k: (_K - 1 - k, 0, 0)),
        pl.BlockSpec((512, 512), lambda k: (0, 0)),
        pl.BlockSpec((128, 512), lambda k: (0, 0)),
        pl.BlockSpec((512, 512), lambda k: (0, 0)),
        pl.BlockSpec((128, 512), lambda k: (0, 0)),
        pl.BlockSpec((256, 1024), lambda k: (0, 0)),
        pl.BlockSpec((1, 1024), lambda k: (0, 0)),
    ]
    out_specs = [
        pl.BlockSpec((_C, _B, 128), lambda k: (k, 0, 0)),
        pl.BlockSpec((_C, _B, 128), lambda k: (_K - 1 - k, 0, 0)),
    ]
    return pl.pallas_call(
        _lstm_body,
        grid=grid,
        in_specs=specs,
        out_specs=out_specs,
        out_shape=[
            jax.ShapeDtypeStruct((_T, _B, 128), jnp.float32),
            jax.ShapeDtypeStruct((_T, _B, 128), jnp.float32),
        ],
        scratch_shapes=[
            pltpu.VMEM((_C, _B, 512), jnp.float32),
            pltpu.VMEM((_C, _B, 512), jnp.float32),
            pltpu.VMEM((2, _B, 256), jnp.float32),
        ],
    )(text_tm, audio_tm, text_tm, audio_tm, wtf, waf, wtb, wab, wblk, bcat)


_PAD = 32
_RT = 256


def _win21(src_ref, base):
    e0 = base - 24
    s1 = src_ref[pl.ds(e0, _RT + 24), :]
    def dbl(a, st, n):
        return a[n:] + a[:a.shape[0] - n], st + n
    s2, st2 = dbl(s1, e0, 1)
    s4, st4 = dbl(s2, st2, 2)
    s8, st8 = dbl(s4, st4, 4)
    s16, st16 = dbl(s8, st8, 8)
    def at(a, st, row0):
        i = row0 - st
        return a[i:i + _RT]
    return (at(s16, st16, base)
            + at(s4, st4, base - 16)
            + at(s1, e0, base - 20))


def _grn_body(hsf_b, hsb_b, idg, out, cur, nxt, acc):
    ntile = _T // _RT
    cur[pl.ds(0, _PAD), :] = jnp.zeros((_PAD, 256), jnp.float32)
    nxt[pl.ds(0, _PAD), :] = jnp.zeros((_PAD, 256), jnp.float32)
    for rt in range(ntile):
        r = rt * _RT
        x = jnp.concatenate([hsf_b[pl.ds(r, _RT), 0, 0, :],
                             hsb_b[pl.ds(r, _RT), 0, 0, :]], axis=1)
        cur[pl.ds(_PAD + r, _RT), :] = x
        acc[pl.ds(r, _RT), :] = x
    src, dst = cur, nxt
    for _ in range(_HOPS):
        for rt in range(ntile):
            r = rt * _RT
            w = _win21(src, _PAD + r)
            nv = w * idg[pl.ds(r, _RT), :]
            dst[pl.ds(_PAD + r, _RT), :] = nv
            acc[pl.ds(r, _RT), :] = acc[pl.ds(r, _RT), :] + nv
        src, dst = dst, src
    for rt in range(ntile):
        r = rt * _RT
        out[pl.ds(r, _RT), 0, 0, :] = acc[pl.ds(r, _RT), :] * 0.25


def _grn(hsf, hsb, idg):
    out = pl.pallas_call(
        _grn_body,
        grid=(_B,),
        in_specs=[
            pl.BlockSpec((_T, 1, 1, 128), lambda b: (0, b, 0, 0)),
            pl.BlockSpec((_T, 1, 1, 128), lambda b: (0, b, 0, 0)),
            pl.BlockSpec((_T, 1), lambda b: (0, 0)),
        ],
        out_specs=pl.BlockSpec((_T, 1, 1, 256), lambda b: (0, b, 0, 0)),
        out_shape=jax.ShapeDtypeStruct((_T, _B, 1, 256), jnp.float32),
        scratch_shapes=[
            pltpu.VMEM((_T + _PAD, 256), jnp.float32),
            pltpu.VMEM((_T + _PAD, 256), jnp.float32),
            pltpu.VMEM((_T, 256), jnp.float32),
        ],
    )(hsf.reshape(_T, _B, 1, 128), hsb.reshape(_T, _B, 1, 128), idg)
    return out.reshape(_T, _B, 256)


_CH = 256
_NCH = _T // _CH
_HALO = 60
_ROWS = _CH + _HALO
_NLANE = 8
_IDGN = 336


def _sc_hop(src, dst, accb, idgv, h):
    r_lo = 20 * (h + 1)
    nj = _NLANE

    def win_init(r, s):
        return tuple(s[j] + src[r, 0, pl.ds(16 * j, 16)] for j in range(nj))

    s = lax.fori_loop(r_lo - 20, r_lo,
                      win_init,
                      tuple(jnp.zeros((16,), jnp.float32) for _ in range(nj)))

    def mk_body(with_acc):
        def body(r, s):
            news = [s[j] + src[r, 0, pl.ds(16 * j, 16)] for j in range(nj)]
            idg = idgv[pl.ds(r, 16)][0]
            for j in range(nj):
                cur = news[j] * idg
                dst[r, 0, pl.ds(16 * j, 16)] = cur
                if with_acc:
                    a = accb[r - _HALO, 0, pl.ds(16 * j, 16)] + cur
                    if h == _HOPS - 1:
                        a = a * (1.0 / (_HOPS + 1))
                    accb[r - _HALO, 0, pl.ds(16 * j, 16)] = a
            return tuple(news[j] - src[r - 20, 0, pl.ds(16 * j, 16)]
                         for j in range(nj))
        return body

    if r_lo < _HALO:
        s = lax.fori_loop(r_lo, _HALO, mk_body(False), s)
    lax.fori_loop(_HALO, _ROWS, mk_body(True), s)


def _grn_sc_body(hsf_hbm, hsb_hbm, idg_hbm, outf_hbm, outb_hbm,
                 abuf, bbuf, accb, idgv):
    wid = lax.axis_index("s") * 2 + lax.axis_index("c")

    for i in range(4):
        src_arr = hsf_hbm if (i % 2 == 0) else hsb_hbm
        out_arr = outf_hbm if (i % 2 == 0) else outb_hbm
        rest = wid + 32 * (i // 2)
        b = rest // _NCH
        tc = rest % _NCH
        start = tc * _CH
        halo0 = jnp.maximum(start - _HALO, 0)

        pltpu.sync_copy(idg_hbm.at[pl.ds(start, _IDGN)], idgv)

        @pl.when(tc > 0)
        def _():
            pltpu.sync_copy(
                src_arr.at[pl.ds(halo0, _HALO), pl.ds(b, 1), :],
                abuf.at[pl.ds(0, _HALO)])

        @pl.when(tc == 0)
        def _():
            def zrow(r, c):
                for j in range(_NLANE):
                    abuf[r, 0, pl.ds(16 * j, 16)] = jnp.zeros((16,),
                                                              jnp.float32)
                return c
            lax.fori_loop(0, _HALO, zrow, 0)

        pltpu.sync_copy(
            src_arr.at[pl.ds(start, _CH), pl.ds(b, 1), :],
            abuf.at[pl.ds(_HALO, _CH)])
        pltpu.sync_copy(
            src_arr.at[pl.ds(start, _CH), pl.ds(b, 1), :], accb)

        _sc_hop(abuf, bbuf, accb, idgv, 0)
        _sc_hop(bbuf, abuf, accb, idgv, 1)
        _sc_hop(abuf, bbuf, accb, idgv, 2)

        pltpu.sync_copy(
            accb, out_arr.at[pl.ds(start, _CH), pl.ds(b, 1), :])


def _grn_sc(hsf, hsb, idg_pad):
    mesh = plsc.VectorSubcoreMesh(core_axis_name="c", subcore_axis_name="s")
    f = functools.partial(
        pl.kernel,
        out_type=[
            jax.ShapeDtypeStruct((_T, _B, 128), jnp.float32),
            jax.ShapeDtypeStruct((_T, _B, 128), jnp.float32),
        ],
        mesh=mesh,
        scratch_types=[
            pltpu.VMEM((_ROWS, 1, 128), jnp.float32),
            pltpu.VMEM((_ROWS, 1, 128), jnp.float32),
            pltpu.VMEM((_CH, 1, 128), jnp.float32),
            pltpu.VMEM((_IDGN,), jnp.float32),
        ],
    )(_grn_sc_body)
    return f(hsf, hsb, idg_pad)


_CF = 256


def _fusion_body(hsf, hsb, grf, grb, wg_l, wg_g, bg, wx, wgr, bfv, wc, bc,
                 out):
    n = _CF * _B
    l = jnp.concatenate([hsf[...].reshape(n, 128), hsb[...].reshape(n, 128)],
                        axis=1)
    g = jnp.concatenate([grf[...].reshape(n, 128), grb[...].reshape(n, 128)],
                        axis=1)
    gate = jax.nn.sigmoid(jnp.dot(l, wg_l[...], precision=_PREC)
                          + jnp.dot(g, wg_g[...], precision=_PREC) + bg[...])
    fused = jnp.tanh(gate * jnp.dot(l, wx[...], precision=_PREC)
                     + (1.0 - gate) * jnp.dot(g, wgr[...], precision=_PREC)
                     + bfv[...])
    out[...] = (jnp.dot(fused, wc[...], precision=_PREC)
                + bc[...]).reshape(_CF, _B, 128)


def _fusion(hsf, hsb, grf, grb, wg_l, wg_g, bg, wx, wgr, bfv, wc, bc):
    m = _T // _CF
    wspec = lambda shp: pl.BlockSpec(shp, lambda k: (0, 0))
    return pl.pallas_call(
        _fusion_body,
        grid=(m,),
        in_specs=[
            pl.BlockSpec((_CF, _B, 128), lambda k: (k, 0, 0)),
            pl.BlockSpec((_CF, _B, 128), lambda k: (k, 0, 0)),
            pl.BlockSpec((_CF, _B, 128), lambda k: (k, 0, 0)),
            pl.BlockSpec((_CF, _B, 128), lambda k: (k, 0, 0)),
            wspec((256, 256)), wspec((256, 256)), wspec((1, 256)),
            wspec((256, 256)), wspec((256, 256)), wspec((1, 256)),
            wspec((256, 128)), wspec((1, 128)),
        ],
        out_specs=pl.BlockSpec((_CF, _B, 128), lambda k: (k, 0, 0)),
        out_shape=jax.ShapeDtypeStruct((_T, _B, 128), jnp.float32),
    )(hsf, hsb, grf, grb, wg_l, wg_g, bg, wx, wgr, bfv, wc, bc)


def kernel(text_embeds, audio_feats, speaker_ids, W_ih_f, W_hh_f, b_f,
           W_ih_b, W_hh_b, b_b, Wg, bg, Wx, Wgr, bf, Wc, bc):
    del speaker_ids
    f32 = jnp.float32

    text_tm = jnp.swapaxes(text_embeds, 0, 1)
    audio_tm = jnp.swapaxes(audio_feats, 0, 1)

    wtf = W_ih_f[:, :512].T
    waf = W_ih_f[:, 512:].T
    wtb = W_ih_b[:, :512].T
    wab = W_ih_b[:, 512:].T
    wblk = jnp.zeros((256, 1024), f32)
    wblk = wblk.at[:128, :512].set(W_hh_f.T)
    wblk = wblk.at[128:, 512:].set(W_hh_b.T)
    bcat = jnp.concatenate([b_f, b_b]).reshape(1, 1024)

    hsf, hsb = _lstm(text_tm, audio_tm, wtf, waf, wtb, wab, wblk, bcat)

    p = jnp.arange(_T + 128, dtype=f32)
    idg_pad = 1.0 / jnp.clip(p - _HALO + 1.0, 1.0, 21.0)
    grf, grb = _grn_sc(hsf, hsb, idg_pad)

    wg_l = Wg[:, :256].T
    wg_g = Wg[:, 256:].T
    wc_pad = jnp.zeros((256, 128), f32).at[:, :7].set(Wc.T)
    bc_pad = jnp.zeros((1, 128), f32).at[0, :7].set(bc)
    out = _fusion(hsf, hsb, grf, grb, wg_l, wg_g, bg.reshape(1, 256),
                  Wx.T, Wgr.T, bf.reshape(1, 256), wc_pad, bc_pad)

    return jnp.swapaxes(out[:, :, :7], 0, 1)


# speedup vs baseline: 3.0099x; 1.0251x over previous
import functools

import jax
import jax.numpy as jnp
from jax import lax
from jax.experimental import pallas as pl
from jax.experimental.pallas import tpu as pltpu
from jax.experimental.pallas import tpu_sc as plsc

_T = 2048
_B = 8
_H = 128
_WIN = 20
_HOPS = 3
_PREC = lax.Precision.DEFAULT
_PREC_R = lax.Precision.DEFAULT


_C = 128
_K = _T // _C


def _lstm_body(text_f, audio_f, text_b, audio_b, wtf, waf, wtb, wab, wblk,
               bcat, hsf, hsb, gxf, gxb, hc):
    k = pl.program_id(0)

    xt_f = text_f[...].reshape(_C * _B, 512)
    xa_f = audio_f[...].reshape(_C * _B, 128)
    gxf[...] = (jnp.dot(xt_f, wtf[...], precision=_PREC)
                + jnp.dot(xa_f, waf[...], precision=_PREC)
                + bcat[:, :512]).reshape(_C, _B, 512)
    xt_b = text_b[...].reshape(_C * _B, 512)
    xa_b = audio_b[...].reshape(_C * _B, 128)
    gxb[...] = (jnp.dot(xt_b, wtb[...], precision=_PREC)
                + jnp.dot(xa_b, wab[...], precision=_PREC)
                + bcat[:, 512:]).reshape(_C, _B, 512)

    @pl.when(k == 0)
    def _():
        hc[...] = jnp.zeros_like(hc)

    h0 = hc[0]
    c0 = hc[1]

    def step(s, carry):
        h, c = carry
        ghh = jnp.dot(h, wblk[...], precision=_PREC_R)
        gf = gxf[s] + ghh[:, :512]
        gb = gxb[_C - 1 - s] + ghh[:, 512:]
        i_f = jax.nn.sigmoid(gf[:, 0:128])
        f_f = jax.nn.sigmoid(gf[:, 128:256])
        g_f = jnp.tanh(gf[:, 256:384])
        o_f = jax.nn.sigmoid(gf[:, 384:512])
        i_b = jax.nn.sigmoid(gb[:, 0:128])
        f_b = jax.nn.sigmoid(gb[:, 128:256])
        g_b = jnp.tanh(gb[:, 256:384])
        o_b = jax.nn.sigmoid(gb[:, 384:512])
        c_f = f_f * c[:, :128] + i_f * g_f
        c_b = f_b * c[:, 128:] + i_b * g_b
        h_f = o_f * jnp.tanh(c_f)
        h_b = o_b * jnp.tanh(c_b)
        hsf[s] = h_f
        hsb[_C - 1 - s] = h_b
        return (jnp.concatenate([h_f, h_b], axis=1),
                jnp.concatenate([c_f, c_b], axis=1))

    h, c = lax.fori_loop(0, _C, step, (h0, c0), unroll=4)
    hc[0] = h
    hc[1] = c


def _lstm(text_tm, audio_tm, wtf, waf, wtb, wab, wblk, bcat):
    grid = (_K,)
    full = lambda *_: tuple(0 for _ in range(2))
    specs = [
        pl.BlockSpec((_C, _B, 512), lambda k: (k, 0, 0)),
        pl.BlockSpec((_C, _B, 128), lambda k: (k, 0, 0)),
        pl.BlockSpec((_C, _B, 512), lambda k: (_K - 1 - k, 0, 0)),
        pl.BlockSpec((_C, _B, 128), lambda k: (_K - 1 - k, 0, 0)),
        pl.BlockSpec((512, 512), lambda k: (0, 0)),
        pl.BlockSpec((128, 512), lambda k: (0, 0)),
        pl.BlockSpec((512, 512), lambda k: (0, 0)),
        pl.BlockSpec((128, 512), lambda k: (0, 0)),
        pl.BlockSpec((256, 1024), lambda k: (0, 0)),
        pl.BlockSpec((1, 1024), lambda k: (0, 0)),
    ]
    out_specs = [
        pl.BlockSpec((_C, _B, 128), lambda k: (k, 0, 0)),
        pl.BlockSpec((_C, _B, 128), lambda k: (_K - 1 - k, 0, 0)),
    ]
    return pl.pallas_call(
        _lstm_body,
        grid=grid,
        in_specs=specs,
        out_specs=out_specs,
        out_shape=[
            jax.ShapeDtypeStruct((_T, _B, 128), jnp.float32),
            jax.ShapeDtypeStruct((_T, _B, 128), jnp.float32),
        ],
        scratch_shapes=[
            pltpu.VMEM((_C, _B, 512), jnp.float32),
            pltpu.VMEM((_C, _B, 512), jnp.float32),
            pltpu.VMEM((2, _B, 256), jnp.float32),
        ],
    )(text_tm, audio_tm, text_tm, audio_tm, wtf, waf, wtb, wab, wblk, bcat)


_PAD = 32
_RT = 256


def _win21(src_ref, base):
    e0 = base - 24
    s1 = src_ref[pl.ds(e0, _RT + 24), :]
    def dbl(a, st, n):
        return a[n:] + a[:a.shape[0] - n], st + n
    s2, st2 = dbl(s1, e0, 1)
    s4, st4 = dbl(s2, st2, 2)
    s8, st8 = dbl(s4, st4, 4)
    s16, st16 = dbl(s8, st8, 8)
    def at(a, st, row0):
        i = row0 - st
        return a[i:i + _RT]
    return (at(s16, st16, base)
            + at(s4, st4, base - 16)
            + at(s1, e0, base - 20))


def _grn_body(hsf_b, hsb_b, idg, out, cur, nxt, acc):
    ntile = _T // _RT
    cur[pl.ds(0, _PAD), :] = jnp.zeros((_PAD, 256), jnp.float32)
    nxt[pl.ds(0, _PAD), :] = jnp.zeros((_PAD, 256), jnp.float32)
    for rt in range(ntile):
        r = rt * _RT
        x = jnp.concatenate([hsf_b[pl.ds(r, _RT), 0, 0, :],
                             hsb_b[pl.ds(r, _RT), 0, 0, :]], axis=1)
        cur[pl.ds(_PAD + r, _RT), :] = x
        acc[pl.ds(r, _RT), :] = x
    src, dst = cur, nxt
    for _ in range(_HOPS):
        for rt in range(ntile):
            r = rt * _RT
            w = _win21(src, _PAD + r)
            nv = w * idg[pl.ds(r, _RT), :]
            dst[pl.ds(_PAD + r, _RT), :] = nv
            acc[pl.ds(r, _RT), :] = acc[pl.ds(r, _RT), :] + nv
        src, dst = dst, src
    for rt in range(ntile):
        r = rt * _RT
        out[pl.ds(r, _RT), 0, 0, :] = acc[pl.ds(r, _RT), :] * 0.25


def _grn(hsf, hsb, idg):
    out = pl.pallas_call(
        _grn_body,
        grid=(_B,),
        in_specs=[
            pl.BlockSpec((_T, 1, 1, 128), lambda b: (0, b, 0, 0)),
            pl.BlockSpec((_T, 1, 1, 128), lambda b: (0, b, 0, 0)),
            pl.BlockSpec((_T, 1), lambda b: (0, 0)),
        ],
        out_specs=pl.BlockSpec((_T, 1, 1, 256), lambda b: (0, b, 0, 0)),
        out_shape=jax.ShapeDtypeStruct((_T, _B, 1, 256), jnp.float32),
        scratch_shapes=[
            pltpu.VMEM((_T + _PAD, 256), jnp.float32),
            pltpu.VMEM((_T + _PAD, 256), jnp.float32),
            pltpu.VMEM((_T, 256), jnp.float32),
        ],
    )(hsf.reshape(_T, _B, 1, 128), hsb.reshape(_T, _B, 1, 128), idg)
    return out.reshape(_T, _B, 256)


_CH = 256
_NCH = _T // _CH
_HALO = 60
_ROWS = _CH + _HALO
_NLANE = 8
_IDGN = 336


def _sc_hop(src, dst, accb, idgv, h):
    r_lo = 20 * (h + 1)
    nj = _NLANE

    def win_init(r, s):
        return tuple(s[j] + src[r, 0, pl.ds(16 * j, 16)] for j in range(nj))

    s = lax.fori_loop(r_lo - 20, r_lo,
                      win_init,
                      tuple(jnp.zeros((16,), jnp.float32) for _ in range(nj)))

    def mk_body(with_acc):
        def body(r, s):
            a_r = [src[r, 0, pl.ds(16 * j, 16)] for j in range(nj)]
            news = [s[j] + a_r[j] for j in range(nj)]
            idg = idgv[pl.ds(r, 16)][0]
            for j in range(nj):
                cur = news[j] * idg
                dst[r, 0, pl.ds(16 * j, 16)] = cur
                if with_acc:
                    if h == 0:
                        a = a_r[j] + cur
                    else:
                        a = accb[r - _HALO, 0, pl.ds(16 * j, 16)] + cur
                        if h == _HOPS - 1:
                            a = a * (1.0 / (_HOPS + 1))
                    accb[r - _HALO, 0, pl.ds(16 * j, 16)] = a
            return tuple(news[j] - src[r - 20, 0, pl.ds(16 * j, 16)]
                         for j in range(nj))
        return body

    if r_lo < _HALO:
        s = lax.fori_loop(r_lo, _HALO, mk_body(False), s)
    lax.fori_loop(_HALO, _ROWS, mk_body(True), s)


def _grn_sc_body(hsf_hbm, hsb_hbm, idg_hbm, outf_hbm, outb_hbm,
                 abuf, bbuf, accb, idgv, sem_idg, sem_main, sem_out):
    wid = lax.axis_index("s") * 2 + lax.axis_index("c")

    out_pend = None
    for i in range(4):
        src_arr = hsf_hbm if (i % 2 == 0) else hsb_hbm
        out_arr = outf_hbm if (i % 2 == 0) else outb_hbm
        rest = wid + 32 * (i // 2)
        b = rest // _NCH
        tc = rest % _NCH
        start = tc * _CH
        halo0 = jnp.maximum(start - _HALO, 0)

        d_idg = pltpu.async_copy(idg_hbm.at[pl.ds(start, _IDGN)], idgv,
                                 sem_idg)
        d_main = pltpu.async_copy(
            src_arr.at[pl.ds(start, _CH), pl.ds(b, 1), :],
            abuf.at[pl.ds(_HALO, _CH)], sem_main)

        @pl.when(tc > 0)
        def _():
            pltpu.sync_copy(
                src_arr.at[pl.ds(halo0, _HALO), pl.ds(b, 1), :],
                abuf.at[pl.ds(0, _HALO)])

        @pl.when(tc == 0)
        def _():
            def zrow(r, c):
                for j in range(_NLANE):
                    abuf[r, 0, pl.ds(16 * j, 16)] = jnp.zeros((16,),
                                                              jnp.float32)
                return c
            lax.fori_loop(0, _HALO, zrow, 0)

        if out_pend is not None:
            out_pend.wait()
        d_idg.wait()
        d_main.wait()

        _sc_hop(abuf, bbuf, accb, idgv, 0)
        _sc_hop(bbuf, abuf, accb, idgv, 1)
        _sc_hop(abuf, bbuf, accb, idgv, 2)

        out_pend = pltpu.async_copy(
            accb, out_arr.at[pl.ds(start, _CH), pl.ds(b, 1), :], sem_out)
    out_pend.wait()


def _grn_sc(hsf, hsb, idg_pad):
    mesh = plsc.VectorSubcoreMesh(core_axis_name="c", subcore_axis_name="s")
    f = functools.partial(
        pl.kernel,
        out_type=[
            jax.ShapeDtypeStruct((_T, _B, 128), jnp.float32),
            jax.ShapeDtypeStruct((_T, _B, 128), jnp.float32),
        ],
        mesh=mesh,
        scratch_types=[
            pltpu.VMEM((_ROWS, 1, 128), jnp.float32),
            pltpu.VMEM((_ROWS, 1, 128), jnp.float32),
            pltpu.VMEM((_CH, 1, 128), jnp.float32),
            pltpu.VMEM((_IDGN,), jnp.float32),
            pltpu.SemaphoreType.DMA,
            pltpu.SemaphoreType.DMA,
            pltpu.SemaphoreType.DMA,
        ],
    )(_grn_sc_body)
    return f(hsf, hsb, idg_pad)


_CF = 256


def _fusion_body(hsf, hsb, grf, grb, wg_l, wg_g, bg, wx, wgr, bfv, wc, bc,
                 out):
    n = _CF * _B
    l = jnp.concatenate([hsf[...].reshape(n, 128), hsb[...].reshape(n, 128)],
                        axis=1)
    g = jnp.concatenate([grf[...].reshape(n, 128), grb[...].reshape(n, 128)],
                        axis=1)
    gate = jax.nn.sigmoid(jnp.dot(l, wg_l[...], precision=_PREC)
                          + jnp.dot(g, wg_g[...], precision=_PREC) + bg[...])
    fused = jnp.tanh(gate * jnp.dot(l, wx[...], precision=_PREC)
                     + (1.0 - gate) * jnp.dot(g, wgr[...], precision=_PREC)
                     + bfv[...])
    out[...] = (jnp.dot(fused, wc[...], precision=_PREC)
                + bc[...]).reshape(_CF, _B, 128)


def _fusion(hsf, hsb, grf, grb, wg_l, wg_g, bg, wx, wgr, bfv, wc, bc):
    m = _T // _CF
    wspec = lambda shp: pl.BlockSpec(shp, lambda k: (0, 0))
    return pl.pallas_call(
        _fusion_body,
        grid=(m,),
        in_specs=[
            pl.BlockSpec((_CF, _B, 128), lambda k: (k, 0, 0)),
            pl.BlockSpec((_CF, _B, 128), lambda k: (k, 0, 0)),
            pl.BlockSpec((_CF, _B, 128), lambda k: (k, 0, 0)),
            pl.BlockSpec((_CF, _B, 128), lambda k: (k, 0, 0)),
            wspec((256, 256)), wspec((256, 256)), wspec((1, 256)),
            wspec((256, 256)), wspec((256, 256)), wspec((1, 256)),
            wspec((256, 128)), wspec((1, 128)),
        ],
        out_specs=pl.BlockSpec((_CF, _B, 128), lambda k: (k, 0, 0)),
        out_shape=jax.ShapeDtypeStruct((_T, _B, 128), jnp.float32),
    )(hsf, hsb, grf, grb, wg_l, wg_g, bg, wx, wgr, bfv, wc, bc)


def kernel(text_embeds, audio_feats, speaker_ids, W_ih_f, W_hh_f, b_f,
           W_ih_b, W_hh_b, b_b, Wg, bg, Wx, Wgr, bf, Wc, bc):
    del speaker_ids
    f32 = jnp.float32

    text_tm = jnp.swapaxes(text_embeds, 0, 1)
    audio_tm = jnp.swapaxes(audio_feats, 0, 1)

    wtf = W_ih_f[:, :512].T
    waf = W_ih_f[:, 512:].T
    wtb = W_ih_b[:, :512].T
    wab = W_ih_b[:, 512:].T
    wblk = jnp.zeros((256, 1024), f32)
    wblk = wblk.at[:128, :512].set(W_hh_f.T)
    wblk = wblk.at[128:, 512:].set(W_hh_b.T)
    bcat = jnp.concatenate([b_f, b_b]).reshape(1, 1024)

    hsf, hsb = _lstm(text_tm, audio_tm, wtf, waf, wtb, wab, wblk, bcat)

    p = jnp.arange(_T + 128, dtype=f32)
    idg_pad = 1.0 / jnp.clip(p - _HALO + 1.0, 1.0, 21.0)
    grf, grb = _grn_sc(hsf, hsb, idg_pad)

    wg_l = Wg[:, :256].T
    wg_g = Wg[:, 256:].T
    wc_pad = jnp.zeros((256, 128), f32).at[:, :7].set(Wc.T)
    bc_pad = jnp.zeros((1, 128), f32).at[0, :7].set(bc)
    out = _fusion(hsf, hsb, grf, grb, wg_l, wg_g, bg.reshape(1, 256),
                  Wx.T, Wgr.T, bf.reshape(1, 256), wc_pad, bc_pad)

    return jnp.swapaxes(out[:, :, :7], 0, 1)


# speedup vs baseline: 3.0367x; 1.0089x over previous
import functools

import jax
import jax.numpy as jnp
from jax import lax
from jax.experimental import pallas as pl
from jax.experimental.pallas import tpu as pltpu
from jax.experimental.pallas import tpu_sc as plsc

_T = 2048
_B = 8
_H = 128
_WIN = 20
_HOPS = 3
_PREC = lax.Precision.DEFAULT
_PREC_R = lax.Precision.DEFAULT


_C = 128
_K = _T // _C


def _lstm_body(text_f, audio_f, text_b, audio_b, wtf, waf, wtb, wab, wblk,
               bcat, hsf, hsb, gxf, gxb, hc):
    k = pl.program_id(0)

    xt_f = text_f[...].reshape(_C * _B, 512)
    xa_f = audio_f[...].reshape(_C * _B, 128)
    gxf[...] = (jnp.dot(xt_f, wtf[...], precision=_PREC)
                + jnp.dot(xa_f, waf[...], precision=_PREC)
                + bcat[:, :512]).reshape(_C, _B, 512)
    xt_b = text_b[...].reshape(_C * _B, 512)
    xa_b = audio_b[...].reshape(_C * _B, 128)
    gxb[...] = (jnp.dot(xt_b, wtb[...], precision=_PREC)
                + jnp.dot(xa_b, wab[...], precision=_PREC)
                + bcat[:, 512:]).reshape(_C, _B, 512)

    @pl.when(k == 0)
    def _():
        hc[...] = jnp.zeros_like(hc)

    h0 = hc[0]
    c0 = hc[1]

    def step(s, carry):
        h, c = carry
        ghh = jnp.dot(h.astype(jnp.bfloat16), wblk[...],
                      preferred_element_type=jnp.float32)
        gf = gxf[s] + ghh[:, :512]
        gb = gxb[_C - 1 - s] + ghh[:, 512:]
        i_f = jax.nn.sigmoid(gf[:, 0:128])
        f_f = jax.nn.sigmoid(gf[:, 128:256])
        g_f = jnp.tanh(gf[:, 256:384])
        o_f = jax.nn.sigmoid(gf[:, 384:512])
        i_b = jax.nn.sigmoid(gb[:, 0:128])
        f_b = jax.nn.sigmoid(gb[:, 128:256])
        g_b = jnp.tanh(gb[:, 256:384])
        o_b = jax.nn.sigmoid(gb[:, 384:512])
        c_f = f_f * c[:, :128] + i_f * g_f
        c_b = f_b * c[:, 128:] + i_b * g_b
        h_f = o_f * jnp.tanh(c_f)
        h_b = o_b * jnp.tanh(c_b)
        hsf[s] = h_f
        hsb[_C - 1 - s] = h_b
        return (jnp.concatenate([h_f, h_b], axis=1),
                jnp.concatenate([c_f, c_b], axis=1))

    h, c = lax.fori_loop(0, _C, step, (h0, c0), unroll=8)
    hc[0] = h
    hc[1] = c


def _lstm(text_tm, audio_tm, wtf, waf, wtb, wab, wblk, bcat):
    grid = (_K,)
    full = lambda *_: tuple(0 for _ in range(2))
    specs = [
        pl.BlockSpec((_C, _B, 512), lambda k: (k, 0, 0)),
        pl.BlockSpec((_C, _B, 128), lambda k: (k, 0, 0)),
        pl.BlockSpec((_C, _B, 512), lambda k: (_K - 1 - k, 0, 0)),
        pl.BlockSpec((_C, _B, 128), lambda k: (_K - 1 - k, 0, 0)),
        pl.BlockSpec((512, 512), lambda k: (0, 0)),
        pl.BlockSpec((128, 512), lambda k: (0, 0)),
        pl.BlockSpec((512, 512), lambda k: (0, 0)),
        pl.BlockSpec((128, 512), lambda k: (0, 0)),
        pl.BlockSpec((256, 1024), lambda k: (0, 0)),
        pl.BlockSpec((1, 1024), lambda k: (0, 0)),
    ]
    out_specs = [
        pl.BlockSpec((_C, _B, 128), lambda k: (k, 0, 0)),
        pl.BlockSpec((_C, _B, 128), lambda k: (_K - 1 - k, 0, 0)),
    ]
    return pl.pallas_call(
        _lstm_body,
        grid=grid,
        in_specs=specs,
        out_specs=out_specs,
        out_shape=[
            jax.ShapeDtypeStruct((_T, _B, 128), jnp.float32),
            jax.ShapeDtypeStruct((_T, _B, 128), jnp.float32),
        ],
        scratch_shapes=[
            pltpu.VMEM((_C, _B, 512), jnp.float32),
            pltpu.VMEM((_C, _B, 512), jnp.float32),
            pltpu.VMEM((2, _B, 256), jnp.float32),
        ],
    )(text_tm, audio_tm, text_tm, audio_tm, wtf, waf, wtb, wab, wblk, bcat)


_PAD = 32
_RT = 256


def _win21(src_ref, base):
    e0 = base - 24
    s1 = src_ref[pl.ds(e0, _RT + 24), :]
    def dbl(a, st, n):
        return a[n:] + a[:a.shape[0] - n], st + n
    s2, st2 = dbl(s1, e0, 1)
    s4, st4 = dbl(s2, st2, 2)
    s8, st8 = dbl(s4, st4, 4)
    s16, st16 = dbl(s8, st8, 8)
    def at(a, st, row0):
        i = row0 - st
        return a[i:i + _RT]
    return (at(s16, st16, base)
            + at(s4, st4, base - 16)
            + at(s1, e0, base - 20))


def _grn_body(hsf_b, hsb_b, idg, out, cur, nxt, acc):
    ntile = _T // _RT
    cur[pl.ds(0, _PAD), :] = jnp.zeros((_PAD, 256), jnp.float32)
    nxt[pl.ds(0, _PAD), :] = jnp.zeros((_PAD, 256), jnp.float32)
    for rt in range(ntile):
        r = rt * _RT
        x = jnp.concatenate([hsf_b[pl.ds(r, _RT), 0, 0, :],
                             hsb_b[pl.ds(r, _RT), 0, 0, :]], axis=1)
        cur[pl.ds(_PAD + r, _RT), :] = x
        acc[pl.ds(r, _RT), :] = x
    src, dst = cur, nxt
    for _ in range(_HOPS):
        for rt in range(ntile):
            r = rt * _RT
            w = _win21(src, _PAD + r)
            nv = w * idg[pl.ds(r, _RT), :]
            dst[pl.ds(_PAD + r, _RT), :] = nv
            acc[pl.ds(r, _RT), :] = acc[pl.ds(r, _RT), :] + nv
        src, dst = dst, src
    for rt in range(ntile):
        r = rt * _RT
        out[pl.ds(r, _RT), 0, 0, :] = acc[pl.ds(r, _RT), :] * 0.25


def _grn(hsf, hsb, idg):
    out = pl.pallas_call(
        _grn_body,
        grid=(_B,),
        in_specs=[
            pl.BlockSpec((_T, 1, 1, 128), lambda b: (0, b, 0, 0)),
            pl.BlockSpec((_T, 1, 1, 128), lambda b: (0, b, 0, 0)),
            pl.BlockSpec((_T, 1), lambda b: (0, 0)),
        ],
        out_specs=pl.BlockSpec((_T, 1, 1, 256), lambda b: (0, b, 0, 0)),
        out_shape=jax.ShapeDtypeStruct((_T, _B, 1, 256), jnp.float32),
        scratch_shapes=[
            pltpu.VMEM((_T + _PAD, 256), jnp.float32),
            pltpu.VMEM((_T + _PAD, 256), jnp.float32),
            pltpu.VMEM((_T, 256), jnp.float32),
        ],
    )(hsf.reshape(_T, _B, 1, 128), hsb.reshape(_T, _B, 1, 128), idg)
    return out.reshape(_T, _B, 256)


_CH = 256
_NCH = _T // _CH
_HALO = 60
_ROWS = _CH + _HALO
_NLANE = 8
_IDGN = 336


def _sc_hop(src, dst, accb, idgv, h):
    r_lo = 20 * (h + 1)
    nj = _NLANE

    def win_init(r, s):
        return tuple(s[j] + src[r, 0, pl.ds(16 * j, 16)] for j in range(nj))

    s = lax.fori_loop(r_lo - 20, r_lo,
                      win_init,
                      tuple(jnp.zeros((16,), jnp.float32) for _ in range(nj)))

    def mk_body(with_acc):
        def body(r, s):
            a_r = [src[r, 0, pl.ds(16 * j, 16)] for j in range(nj)]
            news = [s[j] + a_r[j] for j in range(nj)]
            idg = idgv[pl.ds(r, 16)][0]
            for j in range(nj):
                cur = news[j] * idg
                dst[r, 0, pl.ds(16 * j, 16)] = cur
                if with_acc:
                    if h == 0:
                        a = a_r[j] + cur
                    else:
                        a = accb[r - _HALO, 0, pl.ds(16 * j, 16)] + cur
                        if h == _HOPS - 1:
                            a = a * (1.0 / (_HOPS + 1))
                    accb[r - _HALO, 0, pl.ds(16 * j, 16)] = a
            return tuple(news[j] - src[r - 20, 0, pl.ds(16 * j, 16)]
                         for j in range(nj))
        return body

    if r_lo < _HALO:
        s = lax.fori_loop(r_lo, _HALO, mk_body(False), s)
    lax.fori_loop(_HALO, _ROWS, mk_body(True), s)


def _grn_sc_body(hsf_hbm, hsb_hbm, idg_hbm, outf_hbm, outb_hbm,
                 abuf, bbuf, accb, idgv, sem_idg, sem_main, sem_out):
    wid = lax.axis_index("s") * 2 + lax.axis_index("c")

    out_pend = None
    for i in range(4):
        src_arr = hsf_hbm if (i % 2 == 0) else hsb_hbm
        out_arr = outf_hbm if (i % 2 == 0) else outb_hbm
        rest = wid + 32 * (i // 2)
        b = rest // _NCH
        tc = rest % _NCH
        start = tc * _CH
        halo0 = jnp.maximum(start - _HALO, 0)

        d_idg = pltpu.async_copy(idg_hbm.at[pl.ds(start, _IDGN)], idgv,
                                 sem_idg)
        d_main = pltpu.async_copy(
            src_arr.at[pl.ds(start, _CH), pl.ds(b, 1), :],
            abuf.at[pl.ds(_HALO, _CH)], sem_main)

        @pl.when(tc > 0)
        def _():
            pltpu.sync_copy(
                src_arr.at[pl.ds(halo0, _HALO), pl.ds(b, 1), :],
                abuf.at[pl.ds(0, _HALO)])

        @pl.when(tc == 0)
        def _():
            def zrow(r, c):
                for j in range(_NLANE):
                    abuf[r, 0, pl.ds(16 * j, 16)] = jnp.zeros((16,),
                                                              jnp.float32)
                return c
            lax.fori_loop(0, _HALO, zrow, 0)

        if out_pend is not None:
            out_pend.wait()
        d_idg.wait()
        d_main.wait()

        _sc_hop(abuf, bbuf, accb, idgv, 0)
        _sc_hop(bbuf, abuf, accb, idgv, 1)
        _sc_hop(abuf, bbuf, accb, idgv, 2)

        out_pend = pltpu.async_copy(
            accb, out_arr.at[pl.ds(start, _CH), pl.ds(b, 1), :], sem_out)
    out_pend.wait()


def _grn_sc(hsf, hsb, idg_pad):
    mesh = plsc.VectorSubcoreMesh(core_axis_name="c", subcore_axis_name="s")
    f = functools.partial(
        pl.kernel,
        out_type=[
            jax.ShapeDtypeStruct((_T, _B, 128), jnp.float32),
            jax.ShapeDtypeStruct((_T, _B, 128), jnp.float32),
        ],
        mesh=mesh,
        scratch_types=[
            pltpu.VMEM((_ROWS, 1, 128), jnp.float32),
            pltpu.VMEM((_ROWS, 1, 128), jnp.float32),
            pltpu.VMEM((_CH, 1, 128), jnp.float32),
            pltpu.VMEM((_IDGN,), jnp.float32),
            pltpu.SemaphoreType.DMA,
            pltpu.SemaphoreType.DMA,
            pltpu.SemaphoreType.DMA,
        ],
    )(_grn_sc_body)
    return f(hsf, hsb, idg_pad)


_CF = 256


def _fusion_body(hsf, hsb, grf, grb, wg_l, wg_g, bg, wx, wgr, bfv, wc, bc,
                 out):
    n = _CF * _B
    l = jnp.concatenate([hsf[...].reshape(n, 128), hsb[...].reshape(n, 128)],
                        axis=1)
    g = jnp.concatenate([grf[...].reshape(n, 128), grb[...].reshape(n, 128)],
                        axis=1)
    gate = jax.nn.sigmoid(jnp.dot(l, wg_l[...], precision=_PREC)
                          + jnp.dot(g, wg_g[...], precision=_PREC) + bg[...])
    fused = jnp.tanh(gate * jnp.dot(l, wx[...], precision=_PREC)
                     + (1.0 - gate) * jnp.dot(g, wgr[...], precision=_PREC)
                     + bfv[...])
    out[...] = (jnp.dot(fused, wc[...], precision=_PREC)
                + bc[...]).reshape(_CF, _B, 128)


def _fusion(hsf, hsb, grf, grb, wg_l, wg_g, bg, wx, wgr, bfv, wc, bc):
    m = _T // _CF
    wspec = lambda shp: pl.BlockSpec(shp, lambda k: (0, 0))
    return pl.pallas_call(
        _fusion_body,
        grid=(m,),
        in_specs=[
            pl.BlockSpec((_CF, _B, 128), lambda k: (k, 0, 0)),
            pl.BlockSpec((_CF, _B, 128), lambda k: (k, 0, 0)),
            pl.BlockSpec((_CF, _B, 128), lambda k: (k, 0, 0)),
            pl.BlockSpec((_CF, _B, 128), lambda k: (k, 0, 0)),
            wspec((256, 256)), wspec((256, 256)), wspec((1, 256)),
            wspec((256, 256)), wspec((256, 256)), wspec((1, 256)),
            wspec((256, 128)), wspec((1, 128)),
        ],
        out_specs=pl.BlockSpec((_CF, _B, 128), lambda k: (k, 0, 0)),
        out_shape=jax.ShapeDtypeStruct((_T, _B, 128), jnp.float32),
    )(hsf, hsb, grf, grb, wg_l, wg_g, bg, wx, wgr, bfv, wc, bc)


def kernel(text_embeds, audio_feats, speaker_ids, W_ih_f, W_hh_f, b_f,
           W_ih_b, W_hh_b, b_b, Wg, bg, Wx, Wgr, bf, Wc, bc):
    del speaker_ids
    f32 = jnp.float32

    text_tm = jnp.swapaxes(text_embeds, 0, 1)
    audio_tm = jnp.swapaxes(audio_feats, 0, 1)

    wtf = W_ih_f[:, :512].T
    waf = W_ih_f[:, 512:].T
    wtb = W_ih_b[:, :512].T
    wab = W_ih_b[:, 512:].T
    wblk = jnp.zeros((256, 1024), f32)
    wblk = wblk.at[:128, :512].set(W_hh_f.T)
    wblk = wblk.at[128:, 512:].set(W_hh_b.T)
    wblk = wblk.astype(jnp.bfloat16)
    bcat = jnp.concatenate([b_f, b_b]).reshape(1, 1024)

    hsf, hsb = _lstm(text_tm, audio_tm, wtf, waf, wtb, wab, wblk, bcat)

    p = jnp.arange(_T + 128, dtype=f32)
    idg_pad = 1.0 / jnp.clip(p - _HALO + 1.0, 1.0, 21.0)
    grf, grb = _grn_sc(hsf, hsb, idg_pad)

    wg_l = Wg[:, :256].T
    wg_g = Wg[:, 256:].T
    wc_pad = jnp.zeros((256, 128), f32).at[:, :7].set(Wc.T)
    bc_pad = jnp.zeros((1, 128), f32).at[0, :7].set(bc)
    out = _fusion(hsf, hsb, grf, grb, wg_l, wg_g, bg.reshape(1, 256),
                  Wx.T, Wgr.T, bf.reshape(1, 256), wc_pad, bc_pad)

    return jnp.swapaxes(out[:, :, :7], 0, 1)
